# baseline (speedup 1.0000x reference)
import jax
import jax.numpy as jnp
from jax.experimental import pallas as pl
from jax.experimental.pallas import tpu as pltpu

_N = 4096
_D = 128
_K = 2
_BI = 256
_NBUF = 4
_NSTEP = _N // _BI


def _fused_kernel(h_ref, w_ref, adj_ref, out_ref, hw_ref, buf_ref, sem_ref):
    def copy(step, slot, k):
        return pltpu.make_async_copy(
            adj_ref.at[k, pl.ds(step * _BI, _BI), :],
            buf_ref.at[slot, k],
            sem_ref.at[slot, k],
        )

    for b in range(_NBUF):
        copy(b, b, 0).start()
        copy(b, b, 1).start()

    hw_ref[...] = jnp.dot(h_ref[...], w_ref[...],
                          preferred_element_type=jnp.float32)

    for step in range(_NSTEP):
        slot = step % _NBUF
        copy(step, slot, 0).wait()
        copy(step, slot, 1).wait()
        a = buf_ref[slot, 0] + buf_ref[slot, 1]
        out_ref[step * _BI:(step + 1) * _BI, :] = jnp.dot(
            a, hw_ref[...], preferred_element_type=jnp.float32)
        if step + _NBUF < _NSTEP:
            copy(step + _NBUF, slot, 0).start()
            copy(step + _NBUF, slot, 1).start()


def kernel(h, Adj, weight, bias):
    del bias
    return pl.pallas_call(
        _fused_kernel,
        in_specs=[
            pl.BlockSpec(memory_space=pltpu.MemorySpace.VMEM),
            pl.BlockSpec(memory_space=pltpu.MemorySpace.VMEM),
            pl.BlockSpec(memory_space=pltpu.MemorySpace.HBM),
        ],
        out_specs=pl.BlockSpec(memory_space=pltpu.MemorySpace.VMEM),
        out_shape=jax.ShapeDtypeStruct((_N, _D), jnp.float32),
        scratch_shapes=[
            pltpu.VMEM((_N, _D), jnp.float32),
            pltpu.VMEM((_NBUF, _K, _BI, _N), jnp.float32),
            pltpu.SemaphoreType.DMA((_NBUF, _K)),
        ],
    )(h, weight, Adj)


# speedup vs baseline: 1.0889x; 1.0889x over previous
import jax
import jax.numpy as jnp
from jax.experimental import pallas as pl
from jax.experimental.pallas import tpu as pltpu

_N = 4096
_D = 128
_K = 2
_BI = 256


def _fused_kernel(h_ref, w_ref, adj_ref, out_ref, hw_ref):
    i = pl.program_id(0)

    @pl.when(i == 0)
    def _():
        hw_ref[...] = jnp.dot(h_ref[...], w_ref[...],
                              preferred_element_type=jnp.float32
                              ).astype(jnp.bfloat16)

    a = (adj_ref[0] + adj_ref[1]).astype(jnp.bfloat16)
    out_ref[pl.ds(i * _BI, _BI), :] = jnp.dot(
        a, hw_ref[...], preferred_element_type=jnp.float32)


def kernel(h, Adj, weight, bias):
    del bias
    return pl.pallas_call(
        _fused_kernel,
        grid=(_N // _BI,),
        in_specs=[
            pl.BlockSpec((_N, _D), lambda i: (0, 0)),
            pl.BlockSpec((_D, _D), lambda i: (0, 0)),
            pl.BlockSpec((_K, _BI, _N), lambda i: (0, i, 0)),
        ],
        out_specs=pl.BlockSpec((_N, _D), lambda i: (0, 0)),
        out_shape=jax.ShapeDtypeStruct((_N, _D), jnp.float32),
        scratch_shapes=[pltpu.VMEM((_N, _D), jnp.bfloat16)],
    )(h, weight, Adj)
